# baseline (speedup 1.0000x reference)
import functools

import jax
import jax.numpy as jnp
from jax import lax
from jax.experimental import pallas as pl
from jax.experimental.pallas import tpu as pltpu
from jax.experimental.pallas import tpu_sc as plsc

B, N = 8, 1250
T = B * N
E = 320000
H = 128

_RT = 400
_RE = 2560


def _mm(x, w):
    return jax.lax.dot_general(x, w, (((1,), (0,)), ((), ())),
                               preferred_element_type=jnp.float32)


def _mmb(x, w):
    return jax.lax.dot_general(x.astype(jnp.bfloat16), w.astype(jnp.bfloat16),
                               (((1,), (0,)), ((), ())),
                               preferred_element_type=jnp.float32)


def _full(shape):
    return pl.BlockSpec(shape, lambda i: tuple(0 for _ in shape))


def _rows(blk, width=None):
    if width is None:
        return pl.BlockSpec((blk,), lambda i: (i,))
    return pl.BlockSpec((blk, width), lambda i: (i, 0))


def _towers_body(de, ee, se, ie, dc, ec, dids, eids, *refs):
    (dp_w1, dp_b1, dp_w2, dp_b2, dp_w3, dp_b3,
     ep_w1, ep_b1, ep_w2, ep_b2, ep_w3, ep_b3,
     sp_w1, sp_b1, sp_w2, sp_b2, sp_w3, sp_b3,
     ip_w1, ip_b1, ip_w2, ip_b2, ip_w3, ip_b3,
     cf_w1, cf_b1, cf_w2, cf_b2, cf_w3, cf_b3,
     ef_w1, ef_b1, ef_w2, ef_b2, ef_w3, ef_b3,
     dtab, etab, out) = refs

    def mlp3(x, w1, b1, w2, b2, w3, b3):
        h = jax.nn.relu(_mmb(x, w1[...]) + b1[...])
        h = jax.nn.relu(_mmb(h, w2[...]) + b2[...])
        return _mmb(h, w3[...]) + b3[...]

    gnn = mlp3(de[...], dp_w1, dp_b1, dp_w2, dp_b2, dp_w3, dp_b3)
    esm = mlp3(ee[...], ep_w1, ep_b1, ep_w2, ep_b2, ep_w3, ep_b3)
    gearnet = mlp3(se[...], sp_w1, sp_b1, sp_w2, sp_b2, sp_w3, sp_b3)
    esmif = mlp3(ie[...], ip_w1, ip_b1, ip_w2, ip_b2, ip_w3, ip_b3)

    d1h = (dids[...] == jax.lax.broadcasted_iota(jnp.int32, (_RT, 33), 1)).astype(jnp.float32)
    e1h = (eids[...] == jax.lax.broadcasted_iota(jnp.int32, (_RT, 33), 1)).astype(jnp.float32)
    gnn = gnn + _mm(d1h, dtab[...])
    esm = esm + _mm(e1h, etab[...])

    def conf_mlp(x, w1, b1, w2, b2, w3, b3):
        h = jax.nn.relu(x * w1[...] + b1[...])
        h = jax.nn.relu(_mm(h, w2[...]) + b2[...])
        return _mm(h, w3[...]) + b3[...]

    conf = jax.nn.sigmoid(conf_mlp(dc[...], cf_w1, cf_b1, cf_w2, cf_b2, cf_w3, cf_b3))
    esm_conf = conf_mlp(ec[...], ef_w1, ef_b1, ef_w2, ef_b2, ef_w3, ef_b3)
    out[...] = gnn * conf + esm * esm_conf + gearnet + esmif


def _towers(de, ee, se, ie, dc, ec, dids, eids, params):
    wlist = []
    specs = []
    for name in ('DesignProj', 'ESMProj', 'StructProj', 'ESMIFProj',
                 'DesignConf', 'ESMConf'):
        for (w, b) in params[name]:
            wlist += [w, b]
            specs += [_full(w.shape), _full(b.shape)]
    wlist += [params['DesignEmbedTab'], params['ESMEmbedTab']]
    specs += [_full((33, H)), _full((33, H))]
    grid = T // _RT
    return pl.pallas_call(
        _towers_body,
        grid=(grid,),
        in_specs=[_rows(_RT, 1280), _rows(_RT, 1280), _rows(_RT, 3072),
                  _rows(_RT, 512), _rows(_RT, 1), _rows(_RT, 1),
                  _rows(_RT, 1), _rows(_RT, 1)] + specs,
        out_specs=_rows(_RT, H),
        out_shape=jax.ShapeDtypeStruct((T, H), jnp.float32),
    )(de, ee, se, ie, dc, ec, dids, eids, *wlist)


def _edge_body(he, w1, b1, w2, b2, w3, b3, out):
    h = jax.nn.relu(_mmb(he[...], w1[...]) + b1[...])
    h = jax.nn.relu(_mmb(h, w2[...]) + b2[...])
    out[...] = _mmb(h, w3[...]) + b3[...]


def _edge_embed(h_E, params):
    (w1, b1), (w2, b2), (w3, b3) = params['EdgeEmbed']
    return pl.pallas_call(
        _edge_body,
        grid=(E // _RE,),
        in_specs=[_rows(_RE, 448), _full(w1.shape), _full(b1.shape),
                  _full(w2.shape), _full(b2.shape), _full(w3.shape), _full(b3.shape)],
        out_specs=_rows(_RE, H),
        out_shape=jax.ShapeDtypeStruct((E, H), jnp.float32),
    )(h_E, w1, b1, w2, b2, w3, b3)


def _readout_body(hv, ie_, dc, ro_w, ro_b,
                  m1_w1, m1_b1, m1_w2, m1_b2, m1_w3, m1_b3,
                  m2_w1, m2_b1, m2_w2, m2_b2, m2_w3, m2_b3, out):
    logits = _mm(hv[...], ro_w[...]) + ro_b[...]
    m = jnp.max(logits, axis=-1, keepdims=True)
    s = jnp.sum(jnp.exp(logits - m), axis=-1, keepdims=True)
    confs = 1.0 / s

    def gate(x, w1, b1, w2, b2, w3, b3):
        h = jax.nn.relu(x * w1[...] + b1[...])
        h = jax.nn.relu(_mm(h, w2[...]) + b2[...])
        return jax.nn.sigmoid(_mm(h, w3[...]) + b3[...])

    dcv = dc[...]
    g1 = gate(confs - dcv, m1_w1, m1_b1, m1_w2, m1_b2, m1_w3, m1_b3)
    g2 = gate(dcv - confs, m2_w1, m2_b1, m2_w2, m2_b2, m2_w3, m2_b3)
    hv2 = hv[...] * g1 + ie_[...] * g2
    l2 = _mm(hv2, ro_w[...]) + ro_b[...]
    m2 = jnp.max(l2, axis=-1, keepdims=True)
    lse = m2 + jnp.log(jnp.sum(jnp.exp(l2 - m2), axis=-1, keepdims=True))
    out[...] = l2 - lse


def _readout(h_V, inputs_embeds, dc, params):
    ro_w, ro_b = params['ReadOut']
    wlist = [ro_w, ro_b]
    specs = [_full(ro_w.shape), _full(ro_b.shape)]
    for name in ('MLP1', 'MLP2'):
        for (w, b) in params[name]:
            wlist += [w, b]
            specs += [_full(w.shape), _full(b.shape)]
    return pl.pallas_call(
        _readout_body,
        grid=(T // _RT,),
        in_specs=[_rows(_RT, H), _rows(_RT, H), _rows(_RT, 1)] + specs,
        out_specs=_rows(_RT, 33),
        out_shape=jax.ShapeDtypeStruct((T, 33), jnp.float32),
    )(h_V, inputs_embeds, dc, *wlist)


_GW = 144


def _update_body(hv, p0, p1, awd, aws, hv_out, g_out):
    num = p0[:, :H] + p1[:, :H]
    den = p0[:, H:H + 1] + p1[:, H:H + 1]
    hvn = hv[...] + num / (den + 1e-9)
    hv_out[...] = hvn
    vd = _mm(hvn, awd[...])
    vs = _mm(hvn, aws[...])
    lane = jax.lax.broadcasted_iota(jnp.int32, (_RT, _GW - H), 1)
    tail = vd * (lane == 0).astype(jnp.float32) + vs * (lane == 1).astype(jnp.float32)
    g_out[...] = jnp.concatenate([hvn, tail], axis=1)


def _update_build(h_V, p0, p1, awd, aws):
    return pl.pallas_call(
        _update_body,
        grid=(T // _RT,),
        in_specs=[_rows(_RT, H), _rows(_RT, _GW), _rows(_RT, _GW),
                  _full((H, 1)), _full((H, 1))],
        out_specs=[_rows(_RT, H), _rows(_RT, _GW)],
        out_shape=[jax.ShapeDtypeStruct((T, H), jnp.float32),
                   jax.ShapeDtypeStruct((T, _GW), jnp.float32)],
    )(h_V, p0, p1, awd, aws)


_RE2 = 2000


def _edgecomp_body(gd, gs, he, w1d, w1e, w1s, b1, w2, b2,
                   ew1d, ew1e, ew1s, eb1, ew2, eb2, awe, ab,
                   msg_out, he_out, att_out, m_out):
    i = pl.program_id(0)
    gdv = gd[...]
    gsv = gs[...]
    hev = he[...]
    gdh = gdv[:, :H]
    gsh = gsv[:, :H]
    m1 = _mmb(gdh, w1d[...]) + _mmb(hev, w1e[...]) + _mmb(gsh, w1s[...]) + b1[...]
    msg_out[...] = _mmb(jax.nn.relu(m1), w2[...]) + b2[...]
    e1 = _mmb(gdh, ew1d[...]) + _mmb(hev, ew1e[...]) + _mmb(gsh, ew1s[...]) + eb1[...]
    he_out[...] = hev + _mmb(jax.nn.relu(e1), ew2[...]) + eb2[...]
    att = (gdv[:, H:H + 1] + gsv[:, H + 1:H + 2] + _mm(hev, awe[...]) + ab[0])
    att_out[...] = att
    bm = jnp.full((1, 1), jnp.max(att))

    @pl.when(i == 0)
    def _():
        m_out[...] = bm

    @pl.when(i > 0)
    def _():
        m_out[...] = jnp.maximum(m_out[...], bm)


def _edgecomp(GD, GS, h_Ee, lp):
    (w1, b1), (w2, b2) = lp['msg']
    (ew1, eb1), (ew2, eb2) = lp['edge']
    aw, ab = lp['att']
    args = [GD, GS, h_Ee,
            w1[:H], w1[H:2 * H], w1[2 * H:], b1, w2, b2,
            ew1[:H], ew1[H:2 * H], ew1[2 * H:], eb1, ew2, eb2,
            aw[H:2 * H], ab]
    specs = [_rows(_RE2, _GW), _rows(_RE2, _GW), _rows(_RE2, H)] + \
            [_full(a.shape) for a in args[3:]]
    return pl.pallas_call(
        _edgecomp_body,
        grid=(E // _RE2,),
        in_specs=specs,
        out_specs=[_rows(_RE2, H), _rows(_RE2, H), _rows(_RE2, 1),
                   pl.BlockSpec((1, 1), lambda i: (0, 0))],
        out_shape=[jax.ShapeDtypeStruct((E, H), jnp.float32),
                   jax.ShapeDtypeStruct((E, H), jnp.float32),
                   jax.ShapeDtypeStruct((E, 1), jnp.float32),
                   jax.ShapeDtypeStruct((1, 1), jnp.float32)],
    )(*args)


_NW = 32
_C = 128
_NCH = E // _C
_CPW = -(-_NCH // _NW)
_RPT = T // 16


def _sc_gather(G, dst, src):
    mesh = plsc.VectorSubcoreMesh(core_axis_name="c", subcore_axis_name="s")

    @functools.partial(
        pl.kernel,
        out_type=[jax.ShapeDtypeStruct((E, _GW), jnp.float32),
                  jax.ShapeDtypeStruct((E, _GW), jnp.float32)],
        mesh=mesh,
        compiler_params=pltpu.CompilerParams(use_tc_tiling_on_sc=False),
        scratch_types=[pltpu.VMEM((_C,), jnp.int32),
                       pltpu.VMEM((_C,), jnp.int32),
                       pltpu.VMEM((_C, _GW), jnp.float32),
                       pltpu.VMEM((_C, _GW), jnp.float32),
                       pltpu.SemaphoreType.DMA,
                       pltpu.SemaphoreType.DMA],
    )
    def k(g_hbm, dst_hbm, src_hbm, gd_hbm, gs_hbm, idxd, idxs, rowd, rows_, semd, sems):
        w = lax.axis_index("s") * 2 + lax.axis_index("c")

        def body(j, _):
            cw = w + j * _NW

            @pl.when(cw < _NCH)
            def _():
                base = cw * _C
                pltpu.sync_copy(dst_hbm.at[pl.ds(base, _C)], idxd)
                pltpu.sync_copy(src_hbm.at[pl.ds(base, _C)], idxs)
                cpd = pltpu.async_copy(g_hbm.at[idxd], rowd, semd)
                cps = pltpu.async_copy(g_hbm.at[idxs], rows_, sems)
                cpd.wait()
                cps.wait()
                pltpu.sync_copy(rowd, gd_hbm.at[pl.ds(base, _C)])
                pltpu.sync_copy(rows_, gs_hbm.at[pl.ds(base, _C)])
            return _

        lax.fori_loop(0, _CPW, body, None)

    return k(G, dst, src)


def _sc_scatter(S, dst, zrows):
    mesh = plsc.VectorSubcoreMesh(core_axis_name="c", subcore_axis_name="s")

    @functools.partial(
        pl.kernel,
        out_type=jax.ShapeDtypeStruct((2, T, _GW), jnp.float32),
        mesh=mesh,
        compiler_params=pltpu.CompilerParams(use_tc_tiling_on_sc=False),
        scratch_types=[pltpu.VMEM((_C,), jnp.int32),
                       pltpu.VMEM((_C, _GW), jnp.float32),
                       pltpu.VMEM_SHARED((T, _GW), jnp.float32)],
    )
    def k(s_hbm, dst_hbm, z_hbm, out_hbm, idxv, rowv, acc):
        c = lax.axis_index("c")
        s = lax.axis_index("s")
        w = s * 2 + c
        pltpu.sync_copy(z_hbm, acc.at[pl.ds(s * _RPT, _RPT)])
        plsc.subcore_barrier()

        def body(j, _):
            cw = w + j * _NW

            @pl.when(cw < _NCH)
            def _():
                base = cw * _C
                pltpu.sync_copy(dst_hbm.at[pl.ds(base, _C)], idxv)
                pltpu.sync_copy(s_hbm.at[pl.ds(base, _C)], rowv)
                pltpu.sync_copy(rowv, acc.at[idxv], add=True)
            return _

        lax.fori_loop(0, _CPW, body, None)
        plsc.subcore_barrier()
        pltpu.sync_copy(acc.at[pl.ds(s * _RPT, _RPT)],
                        out_hbm.at[c, pl.ds(s * _RPT, _RPT)])

    return k(S, dst, zrows)


def _scale_body(msg, att, m, s_out):
    w = jnp.exp(att[...] - m[0, 0])
    lane = jax.lax.broadcasted_iota(jnp.int32, (_RE2, _GW - H), 1)
    tail = w * (lane == 0).astype(jnp.float32)
    s_out[...] = jnp.concatenate([msg[...] * w, tail], axis=1)


def _scale(msg, att, M):
    return pl.pallas_call(
        _scale_body,
        grid=(E // _RE2,),
        in_specs=[_rows(_RE2, H), _rows(_RE2, 1), pl.BlockSpec((1, 1), lambda i: (0, 0))],
        out_specs=_rows(_RE2, _GW),
        out_shape=jax.ShapeDtypeStruct((E, _GW), jnp.float32),
    )(msg, att, M)


def kernel(design_embed, esm_embed, struct_embed, esmif_embed, design_confs, esm_confs, h_E, params, design_pred_ids, esm_pred_ids, E_idx, attention_mask, batch_id):
    de = design_embed.reshape(T, -1)
    ee = esm_embed.reshape(T, -1)
    se = struct_embed.reshape(T, -1)
    ie = esmif_embed.reshape(T, -1)
    dc = design_confs.reshape(T, 1)
    ec = esm_confs.reshape(T, 1)
    dids = design_pred_ids.reshape(T, 1).astype(jnp.int32)
    eids = esm_pred_ids.reshape(T, 1).astype(jnp.int32)

    inputs_embeds = _towers(de, ee, se, ie, dc, ec, dids, eids, params)
    h_V = inputs_embeds
    h_Ee = _edge_embed(h_E, params)

    src = E_idx[0].astype(jnp.int32)
    dst = E_idx[1].astype(jnp.int32)
    zerosP = jnp.zeros((T, _GW), jnp.float32)
    zrows = jnp.zeros((_RPT, _GW), jnp.float32)
    p0, p1 = zerosP, zerosP
    layers = params['layers']
    for li, lp in enumerate(layers):
        aw = lp['att'][0]
        h_V, G = _update_build(h_V, p0, p1, aw[:H], aw[2 * H:])
        GD, GS = _sc_gather(G, dst, src)
        msg, h_Ee, att, M = _edgecomp(GD, GS, h_Ee, lp)
        S = _scale(msg, att, M)
        P = _sc_scatter(S, dst, zrows)
        p0, p1 = P[0], P[1]
    zw = jnp.zeros((H, 1), jnp.float32)
    h_V, _ = _update_build(h_V, p0, p1, zw, zw)

    logp = _readout(h_V, inputs_embeds, dc, params)
    return logp.reshape(B, N, 33)


# speedup vs baseline: 1.2709x; 1.2709x over previous
import functools

import jax
import jax.numpy as jnp
from jax import lax
from jax.experimental import pallas as pl
from jax.experimental.pallas import tpu as pltpu
from jax.experimental.pallas import tpu_sc as plsc

B, N = 8, 1250
T = B * N
E = 320000
H = 128

_RT = 400
_RE = 2560


def _mm(x, w):
    return jax.lax.dot_general(x, w, (((1,), (0,)), ((), ())),
                               preferred_element_type=jnp.float32)


def _mmb(x, w):
    return jax.lax.dot_general(x.astype(jnp.bfloat16), w.astype(jnp.bfloat16),
                               (((1,), (0,)), ((), ())),
                               preferred_element_type=jnp.float32)


def _full(shape):
    return pl.BlockSpec(shape, lambda i: tuple(0 for _ in shape))


def _rows(blk, width=None):
    if width is None:
        return pl.BlockSpec((blk,), lambda i: (i,))
    return pl.BlockSpec((blk, width), lambda i: (i, 0))


def _towers_body(de, ee, se, ie, dc, ec, dids, eids, *refs):
    (dp_w1, dp_b1, dp_w2, dp_b2, dp_w3, dp_b3,
     ep_w1, ep_b1, ep_w2, ep_b2, ep_w3, ep_b3,
     sp_w1, sp_b1, sp_w2, sp_b2, sp_w3, sp_b3,
     ip_w1, ip_b1, ip_w2, ip_b2, ip_w3, ip_b3,
     cf_w1, cf_b1, cf_w2, cf_b2, cf_w3, cf_b3,
     ef_w1, ef_b1, ef_w2, ef_b2, ef_w3, ef_b3,
     dtab, etab, out) = refs

    def mlp3(x, w1, b1, w2, b2, w3, b3):
        h = jax.nn.relu(_mmb(x, w1[...]) + b1[...])
        h = jax.nn.relu(_mmb(h, w2[...]) + b2[...])
        return _mmb(h, w3[...]) + b3[...]

    gnn = mlp3(de[...], dp_w1, dp_b1, dp_w2, dp_b2, dp_w3, dp_b3)
    esm = mlp3(ee[...], ep_w1, ep_b1, ep_w2, ep_b2, ep_w3, ep_b3)
    gearnet = mlp3(se[...], sp_w1, sp_b1, sp_w2, sp_b2, sp_w3, sp_b3)
    esmif = mlp3(ie[...], ip_w1, ip_b1, ip_w2, ip_b2, ip_w3, ip_b3)

    d1h = (dids[...] == jax.lax.broadcasted_iota(jnp.int32, (_RT, 33), 1)).astype(jnp.float32)
    e1h = (eids[...] == jax.lax.broadcasted_iota(jnp.int32, (_RT, 33), 1)).astype(jnp.float32)
    gnn = gnn + _mm(d1h, dtab[...])
    esm = esm + _mm(e1h, etab[...])

    def conf_mlp(x, w1, b1, w2, b2, w3, b3):
        h = jax.nn.relu(x * w1[...] + b1[...])
        h = jax.nn.relu(_mm(h, w2[...]) + b2[...])
        return _mm(h, w3[...]) + b3[...]

    conf = jax.nn.sigmoid(conf_mlp(dc[...], cf_w1, cf_b1, cf_w2, cf_b2, cf_w3, cf_b3))
    esm_conf = conf_mlp(ec[...], ef_w1, ef_b1, ef_w2, ef_b2, ef_w3, ef_b3)
    out[...] = gnn * conf + esm * esm_conf + gearnet + esmif


def _towers(de, ee, se, ie, dc, ec, dids, eids, params):
    wlist = []
    specs = []
    for name in ('DesignProj', 'ESMProj', 'StructProj', 'ESMIFProj',
                 'DesignConf', 'ESMConf'):
        for (w, b) in params[name]:
            wlist += [w, b]
            specs += [_full(w.shape), _full(b.shape)]
    wlist += [params['DesignEmbedTab'], params['ESMEmbedTab']]
    specs += [_full((33, H)), _full((33, H))]
    grid = T // _RT
    return pl.pallas_call(
        _towers_body,
        grid=(grid,),
        in_specs=[_rows(_RT, 1280), _rows(_RT, 1280), _rows(_RT, 3072),
                  _rows(_RT, 512), _rows(_RT, 1), _rows(_RT, 1),
                  _rows(_RT, 1), _rows(_RT, 1)] + specs,
        out_specs=_rows(_RT, H),
        out_shape=jax.ShapeDtypeStruct((T, H), jnp.float32),
    )(de, ee, se, ie, dc, ec, dids, eids, *wlist)


def _edge_body(he, w1, b1, w2, b2, w3, b3, out):
    h = jax.nn.relu(_mmb(he[...], w1[...]) + b1[...])
    h = jax.nn.relu(_mmb(h, w2[...]) + b2[...])
    out[...] = _mmb(h, w3[...]) + b3[...]


def _edge_embed(h_E, params):
    (w1, b1), (w2, b2), (w3, b3) = params['EdgeEmbed']
    return pl.pallas_call(
        _edge_body,
        grid=(E // _RE,),
        in_specs=[_rows(_RE, 448), _full(w1.shape), _full(b1.shape),
                  _full(w2.shape), _full(b2.shape), _full(w3.shape), _full(b3.shape)],
        out_specs=_rows(_RE, H),
        out_shape=jax.ShapeDtypeStruct((E, H), jnp.float32),
    )(h_E, w1, b1, w2, b2, w3, b3)


def _readout_body(hv, ie_, dc, ro_w, ro_b,
                  m1_w1, m1_b1, m1_w2, m1_b2, m1_w3, m1_b3,
                  m2_w1, m2_b1, m2_w2, m2_b2, m2_w3, m2_b3, out):
    logits = _mm(hv[...], ro_w[...]) + ro_b[...]
    m = jnp.max(logits, axis=-1, keepdims=True)
    s = jnp.sum(jnp.exp(logits - m), axis=-1, keepdims=True)
    confs = 1.0 / s

    def gate(x, w1, b1, w2, b2, w3, b3):
        h = jax.nn.relu(x * w1[...] + b1[...])
        h = jax.nn.relu(_mm(h, w2[...]) + b2[...])
        return jax.nn.sigmoid(_mm(h, w3[...]) + b3[...])

    dcv = dc[...]
    g1 = gate(confs - dcv, m1_w1, m1_b1, m1_w2, m1_b2, m1_w3, m1_b3)
    g2 = gate(dcv - confs, m2_w1, m2_b1, m2_w2, m2_b2, m2_w3, m2_b3)
    hv2 = hv[...] * g1 + ie_[...] * g2
    l2 = _mm(hv2, ro_w[...]) + ro_b[...]
    m2 = jnp.max(l2, axis=-1, keepdims=True)
    lse = m2 + jnp.log(jnp.sum(jnp.exp(l2 - m2), axis=-1, keepdims=True))
    out[...] = l2 - lse


def _readout(h_V, inputs_embeds, dc, params):
    ro_w, ro_b = params['ReadOut']
    wlist = [ro_w, ro_b]
    specs = [_full(ro_w.shape), _full(ro_b.shape)]
    for name in ('MLP1', 'MLP2'):
        for (w, b) in params[name]:
            wlist += [w, b]
            specs += [_full(w.shape), _full(b.shape)]
    return pl.pallas_call(
        _readout_body,
        grid=(T // _RT,),
        in_specs=[_rows(_RT, H), _rows(_RT, H), _rows(_RT, 1)] + specs,
        out_specs=_rows(_RT, 33),
        out_shape=jax.ShapeDtypeStruct((T, 33), jnp.float32),
    )(h_V, inputs_embeds, dc, *wlist)


_GW = 144


def _update_body(hv, p0, p1, hv_out):
    num = p0[:, :H] + p1[:, :H]
    den = p0[:, H:H + 1] + p1[:, H:H + 1]
    hv_out[...] = hv[...] + num / (den + 1e-9)


def _update_build(h_V, p0, p1):
    return pl.pallas_call(
        _update_body,
        grid=(T // _RT,),
        in_specs=[_rows(_RT, H), _rows(_RT, _GW), _rows(_RT, _GW)],
        out_specs=_rows(_RT, H),
        out_shape=jax.ShapeDtypeStruct((T, H), jnp.float32),
    )(h_V, p0, p1)


_RE2 = 2000


def _edgecomp_body(gd, gs, he, w1d, w1e, w1s, b1, w2, b2,
                   ew1d, ew1e, ew1s, eb1, ew2, eb2, awd, awe, aws, ab,
                   msg_out, he_out, att_out, m_out):
    i = pl.program_id(0)
    gdh = gd[...]
    gsh = gs[...]
    hev = he[...]
    m1 = _mmb(gdh, w1d[...]) + _mmb(hev, w1e[...]) + _mmb(gsh, w1s[...]) + b1[...]
    msg_out[...] = _mmb(jax.nn.relu(m1), w2[...]) + b2[...]
    e1 = _mmb(gdh, ew1d[...]) + _mmb(hev, ew1e[...]) + _mmb(gsh, ew1s[...]) + eb1[...]
    he_out[...] = hev + _mmb(jax.nn.relu(e1), ew2[...]) + eb2[...]
    att = (_mm(gdh, awd[...]) + _mm(hev, awe[...]) + _mm(gsh, aws[...]) + ab[0])
    att_out[...] = att
    bm = jnp.full((1, 1), jnp.max(att))

    @pl.when(i == 0)
    def _():
        m_out[...] = bm

    @pl.when(i > 0)
    def _():
        m_out[...] = jnp.maximum(m_out[...], bm)


def _edgecomp(GD, GS, h_Ee, lp):
    (w1, b1), (w2, b2) = lp['msg']
    (ew1, eb1), (ew2, eb2) = lp['edge']
    aw, ab = lp['att']
    args = [GD, GS, h_Ee,
            w1[:H], w1[H:2 * H], w1[2 * H:], b1, w2, b2,
            ew1[:H], ew1[H:2 * H], ew1[2 * H:], eb1, ew2, eb2,
            aw[:H], aw[H:2 * H], aw[2 * H:], ab]
    specs = [_rows(_RE2, H), _rows(_RE2, H), _rows(_RE2, H)] + \
            [_full(a.shape) for a in args[3:]]
    return pl.pallas_call(
        _edgecomp_body,
        grid=(E // _RE2,),
        in_specs=specs,
        out_specs=[_rows(_RE2, H), _rows(_RE2, H), _rows(_RE2, 1),
                   pl.BlockSpec((1, 1), lambda i: (0, 0))],
        out_shape=[jax.ShapeDtypeStruct((E, H), jnp.float32),
                   jax.ShapeDtypeStruct((E, H), jnp.float32),
                   jax.ShapeDtypeStruct((E, 1), jnp.float32),
                   jax.ShapeDtypeStruct((1, 1), jnp.float32)],
    )(*args)


_NW = 32
_C = 128
_NCH = E // _C
_CPW = -(-_NCH // _NW)
_RPT = T // 16


def _sc_gather(G, dst, src):
    mesh = plsc.VectorSubcoreMesh(core_axis_name="c", subcore_axis_name="s")

    @functools.partial(
        pl.kernel,
        out_type=[jax.ShapeDtypeStruct((E, H), jnp.float32),
                  jax.ShapeDtypeStruct((E, H), jnp.float32)],
        mesh=mesh,
        compiler_params=pltpu.CompilerParams(use_tc_tiling_on_sc=True),
        scratch_types=[pltpu.VMEM((_C,), jnp.int32),
                       pltpu.VMEM((_C,), jnp.int32),
                       pltpu.VMEM((_C, H), jnp.float32),
                       pltpu.VMEM((_C, H), jnp.float32),
                       pltpu.SemaphoreType.DMA,
                       pltpu.SemaphoreType.DMA],
    )
    def k(g_hbm, dst_hbm, src_hbm, gd_hbm, gs_hbm, idxd, idxs, rowd, rows_, semd, sems):
        w = lax.axis_index("s") * 2 + lax.axis_index("c")

        def body(j, _):
            cw = w + j * _NW

            @pl.when(cw < _NCH)
            def _():
                base = cw * _C
                pltpu.sync_copy(dst_hbm.at[pl.ds(base, _C)], idxd)
                pltpu.sync_copy(src_hbm.at[pl.ds(base, _C)], idxs)
                cpd = pltpu.async_copy(g_hbm.at[idxd], rowd, semd)
                cps = pltpu.async_copy(g_hbm.at[idxs], rows_, sems)
                cpd.wait()
                cps.wait()
                pltpu.sync_copy(rowd, gd_hbm.at[pl.ds(base, _C)])
                pltpu.sync_copy(rows_, gs_hbm.at[pl.ds(base, _C)])
            return _

        lax.fori_loop(0, _CPW, body, None)

    return k(G, dst, src)


def _sc_scatter(S, dst, zrows):
    mesh = plsc.VectorSubcoreMesh(core_axis_name="c", subcore_axis_name="s")

    @functools.partial(
        pl.kernel,
        out_type=jax.ShapeDtypeStruct((2, T, _GW), jnp.float32),
        mesh=mesh,
        compiler_params=pltpu.CompilerParams(use_tc_tiling_on_sc=False),
        scratch_types=[pltpu.VMEM((_C,), jnp.int32),
                       pltpu.VMEM((_C, _GW), jnp.float32),
                       pltpu.VMEM_SHARED((T, _GW), jnp.float32)],
    )
    def k(s_hbm, dst_hbm, z_hbm, out_hbm, idxv, rowv, acc):
        c = lax.axis_index("c")
        s = lax.axis_index("s")
        w = s * 2 + c
        pltpu.sync_copy(z_hbm, acc.at[pl.ds(s * _RPT, _RPT)])
        plsc.subcore_barrier()

        def body(j, _):
            cw = w + j * _NW

            @pl.when(cw < _NCH)
            def _():
                base = cw * _C
                pltpu.sync_copy(dst_hbm.at[pl.ds(base, _C)], idxv)
                pltpu.sync_copy(s_hbm.at[pl.ds(base, _C)], rowv)
                pltpu.sync_copy(rowv, acc.at[idxv], add=True)
            return _

        lax.fori_loop(0, _CPW, body, None)
        plsc.subcore_barrier()
        pltpu.sync_copy(acc.at[pl.ds(s * _RPT, _RPT)],
                        out_hbm.at[c, pl.ds(s * _RPT, _RPT)])

    return k(S, dst, zrows)


def _scale_body(msg, att, m, s_out):
    w = jnp.exp(att[...] - m[0, 0])
    lane = jax.lax.broadcasted_iota(jnp.int32, (_RE2, _GW - H), 1)
    tail = w * (lane == 0).astype(jnp.float32)
    s_out[...] = jnp.concatenate([msg[...] * w, tail], axis=1)


def _scale(msg, att, M):
    return pl.pallas_call(
        _scale_body,
        grid=(E // _RE2,),
        in_specs=[_rows(_RE2, H), _rows(_RE2, 1), pl.BlockSpec((1, 1), lambda i: (0, 0))],
        out_specs=_rows(_RE2, _GW),
        out_shape=jax.ShapeDtypeStruct((E, _GW), jnp.float32),
    )(msg, att, M)


def kernel(design_embed, esm_embed, struct_embed, esmif_embed, design_confs, esm_confs, h_E, params, design_pred_ids, esm_pred_ids, E_idx, attention_mask, batch_id):
    de = design_embed.reshape(T, -1)
    ee = esm_embed.reshape(T, -1)
    se = struct_embed.reshape(T, -1)
    ie = esmif_embed.reshape(T, -1)
    dc = design_confs.reshape(T, 1)
    ec = esm_confs.reshape(T, 1)
    dids = design_pred_ids.reshape(T, 1).astype(jnp.int32)
    eids = esm_pred_ids.reshape(T, 1).astype(jnp.int32)

    inputs_embeds = _towers(de, ee, se, ie, dc, ec, dids, eids, params)
    h_V = inputs_embeds
    h_Ee = _edge_embed(h_E, params)

    src = E_idx[0].astype(jnp.int32)
    dst = E_idx[1].astype(jnp.int32)
    zerosP = jnp.zeros((T, _GW), jnp.float32)
    zrows = jnp.zeros((_RPT, _GW), jnp.float32)
    p0, p1 = zerosP, zerosP
    layers = params['layers']
    for li, lp in enumerate(layers):
        h_V = _update_build(h_V, p0, p1)
        GD, GS = _sc_gather(h_V, dst, src)
        msg, h_Ee, att, M = _edgecomp(GD, GS, h_Ee, lp)
        S = _scale(msg, att, M)
        P = _sc_scatter(S, dst, zrows)
        p0, p1 = P[0], P[1]
    h_V = _update_build(h_V, p0, p1)

    logp = _readout(h_V, inputs_embeds, dc, params)
    return logp.reshape(B, N, 33)


# speedup vs baseline: 1.3611x; 1.0710x over previous
import functools

import jax
import jax.numpy as jnp
from jax import lax
from jax.experimental import pallas as pl
from jax.experimental.pallas import tpu as pltpu
from jax.experimental.pallas import tpu_sc as plsc

B, N = 8, 1250
T = B * N
E = 320000
H = 128

_RT = 400
_RE = 2560


def _mm(x, w):
    return jax.lax.dot_general(x, w, (((1,), (0,)), ((), ())),
                               preferred_element_type=jnp.float32)


def _mmb(x, w):
    return jax.lax.dot_general(x.astype(jnp.bfloat16), w.astype(jnp.bfloat16),
                               (((1,), (0,)), ((), ())),
                               preferred_element_type=jnp.float32)


def _full(shape):
    return pl.BlockSpec(shape, lambda i: tuple(0 for _ in shape))


def _rows(blk, width=None):
    if width is None:
        return pl.BlockSpec((blk,), lambda i: (i,))
    return pl.BlockSpec((blk, width), lambda i: (i, 0))


def _towers_body(de, ee, se, ie, dc, ec, dids, eids, *refs):
    (dp_w1, dp_b1, dp_w2, dp_b2, dp_w3, dp_b3,
     ep_w1, ep_b1, ep_w2, ep_b2, ep_w3, ep_b3,
     sp_w1, sp_b1, sp_w2, sp_b2, sp_w3, sp_b3,
     ip_w1, ip_b1, ip_w2, ip_b2, ip_w3, ip_b3,
     cf_w1, cf_b1, cf_w2, cf_b2, cf_w3, cf_b3,
     ef_w1, ef_b1, ef_w2, ef_b2, ef_w3, ef_b3,
     dtab, etab, out) = refs

    def mlp3(x, w1, b1, w2, b2, w3, b3):
        h = jax.nn.relu(_mmb(x, w1[...]) + b1[...])
        h = jax.nn.relu(_mmb(h, w2[...]) + b2[...])
        return _mmb(h, w3[...]) + b3[...]

    gnn = mlp3(de[...], dp_w1, dp_b1, dp_w2, dp_b2, dp_w3, dp_b3)
    esm = mlp3(ee[...], ep_w1, ep_b1, ep_w2, ep_b2, ep_w3, ep_b3)
    gearnet = mlp3(se[...], sp_w1, sp_b1, sp_w2, sp_b2, sp_w3, sp_b3)
    esmif = mlp3(ie[...], ip_w1, ip_b1, ip_w2, ip_b2, ip_w3, ip_b3)

    d1h = (dids[...] == jax.lax.broadcasted_iota(jnp.int32, (_RT, 33), 1)).astype(jnp.float32)
    e1h = (eids[...] == jax.lax.broadcasted_iota(jnp.int32, (_RT, 33), 1)).astype(jnp.float32)
    gnn = gnn + _mm(d1h, dtab[...])
    esm = esm + _mm(e1h, etab[...])

    def conf_mlp(x, w1, b1, w2, b2, w3, b3):
        h = jax.nn.relu(x * w1[...] + b1[...])
        h = jax.nn.relu(_mm(h, w2[...]) + b2[...])
        return _mm(h, w3[...]) + b3[...]

    conf = jax.nn.sigmoid(conf_mlp(dc[...], cf_w1, cf_b1, cf_w2, cf_b2, cf_w3, cf_b3))
    esm_conf = conf_mlp(ec[...], ef_w1, ef_b1, ef_w2, ef_b2, ef_w3, ef_b3)
    out[...] = gnn * conf + esm * esm_conf + gearnet + esmif


def _towers(de, ee, se, ie, dc, ec, dids, eids, params):
    wlist = []
    specs = []
    for name in ('DesignProj', 'ESMProj', 'StructProj', 'ESMIFProj',
                 'DesignConf', 'ESMConf'):
        for (w, b) in params[name]:
            wlist += [w, b]
            specs += [_full(w.shape), _full(b.shape)]
    wlist += [params['DesignEmbedTab'], params['ESMEmbedTab']]
    specs += [_full((33, H)), _full((33, H))]
    grid = T // _RT
    return pl.pallas_call(
        _towers_body,
        grid=(grid,),
        in_specs=[_rows(_RT, 1280), _rows(_RT, 1280), _rows(_RT, 3072),
                  _rows(_RT, 512), _rows(_RT, 1), _rows(_RT, 1),
                  _rows(_RT, 1), _rows(_RT, 1)] + specs,
        out_specs=_rows(_RT, H),
        out_shape=jax.ShapeDtypeStruct((T, H), jnp.float32),
    )(de, ee, se, ie, dc, ec, dids, eids, *wlist)


def _edge_body(he, w1, b1, w2, b2, w3, b3, out):
    h = jax.nn.relu(_mmb(he[...], w1[...]) + b1[...])
    h = jax.nn.relu(_mmb(h, w2[...]) + b2[...])
    out[...] = _mmb(h, w3[...]) + b3[...]


def _edge_embed(h_E, params):
    (w1, b1), (w2, b2), (w3, b3) = params['EdgeEmbed']
    return pl.pallas_call(
        _edge_body,
        grid=(E // _RE,),
        in_specs=[_rows(_RE, 448), _full(w1.shape), _full(b1.shape),
                  _full(w2.shape), _full(b2.shape), _full(w3.shape), _full(b3.shape)],
        out_specs=_rows(_RE, H),
        out_shape=jax.ShapeDtypeStruct((E, H), jnp.float32),
    )(h_E, w1, b1, w2, b2, w3, b3)


def _readout_body(hv, ie_, dc, ro_w, ro_b,
                  m1_w1, m1_b1, m1_w2, m1_b2, m1_w3, m1_b3,
                  m2_w1, m2_b1, m2_w2, m2_b2, m2_w3, m2_b3, out):
    logits = _mm(hv[...], ro_w[...]) + ro_b[...]
    m = jnp.max(logits, axis=-1, keepdims=True)
    s = jnp.sum(jnp.exp(logits - m), axis=-1, keepdims=True)
    confs = 1.0 / s

    def gate(x, w1, b1, w2, b2, w3, b3):
        h = jax.nn.relu(x * w1[...] + b1[...])
        h = jax.nn.relu(_mm(h, w2[...]) + b2[...])
        return jax.nn.sigmoid(_mm(h, w3[...]) + b3[...])

    dcv = dc[...]
    g1 = gate(confs - dcv, m1_w1, m1_b1, m1_w2, m1_b2, m1_w3, m1_b3)
    g2 = gate(dcv - confs, m2_w1, m2_b1, m2_w2, m2_b2, m2_w3, m2_b3)
    hv2 = hv[...] * g1 + ie_[...] * g2
    l2 = _mm(hv2, ro_w[...]) + ro_b[...]
    m2 = jnp.max(l2, axis=-1, keepdims=True)
    lse = m2 + jnp.log(jnp.sum(jnp.exp(l2 - m2), axis=-1, keepdims=True))
    out[...] = l2 - lse


def _readout(h_V, inputs_embeds, dc, params):
    ro_w, ro_b = params['ReadOut']
    wlist = [ro_w, ro_b]
    specs = [_full(ro_w.shape), _full(ro_b.shape)]
    for name in ('MLP1', 'MLP2'):
        for (w, b) in params[name]:
            wlist += [w, b]
            specs += [_full(w.shape), _full(b.shape)]
    return pl.pallas_call(
        _readout_body,
        grid=(T // _RT,),
        in_specs=[_rows(_RT, H), _rows(_RT, H), _rows(_RT, 1)] + specs,
        out_specs=_rows(_RT, 33),
        out_shape=jax.ShapeDtypeStruct((T, 33), jnp.float32),
    )(h_V, inputs_embeds, dc, *wlist)


_GW = 144


def _update_body(hv, p0, p1, pw0, pw1, hv_out):
    num = p0[...] + p1[...]
    den = pw0[:, 0:1] + pw1[:, 0:1]
    hv_out[...] = hv[...] + num / (den + 1e-9)


def _update_build(h_V, p0, p1, pw0, pw1):
    return pl.pallas_call(
        _update_body,
        grid=(T // _RT,),
        in_specs=[_rows(_RT, H), _rows(_RT, H), _rows(_RT, H),
                  _rows(_RT, 16), _rows(_RT, 16)],
        out_specs=_rows(_RT, H),
        out_shape=jax.ShapeDtypeStruct((T, H), jnp.float32),
    )(h_V, p0, p1, pw0, pw1)


_RE2 = 2000


def _edgecomp_body(gd, gs, he, w1d, w1e, w1s, b1, w2, b2,
                   ew1d, ew1e, ew1s, eb1, ew2, eb2, awd, awe, aws, ab,
                   msg_out, he_out, att_out, m_out):
    i = pl.program_id(0)
    gdh = gd[...]
    gsh = gs[...]
    hev = he[...]
    m1 = _mmb(gdh, w1d[...]) + _mmb(hev, w1e[...]) + _mmb(gsh, w1s[...]) + b1[...]
    msg_out[...] = _mmb(jax.nn.relu(m1), w2[...]) + b2[...]
    e1 = _mmb(gdh, ew1d[...]) + _mmb(hev, ew1e[...]) + _mmb(gsh, ew1s[...]) + eb1[...]
    he_out[...] = hev + _mmb(jax.nn.relu(e1), ew2[...]) + eb2[...]
    att = (_mm(gdh, awd[...]) + _mm(hev, awe[...]) + _mm(gsh, aws[...]) + ab[0])
    att_out[...] = att
    bm = jnp.full((1, 1), jnp.max(att))

    @pl.when(i == 0)
    def _():
        m_out[...] = bm

    @pl.when(i > 0)
    def _():
        m_out[...] = jnp.maximum(m_out[...], bm)


def _edgecomp(GD, GS, h_Ee, lp):
    (w1, b1), (w2, b2) = lp['msg']
    (ew1, eb1), (ew2, eb2) = lp['edge']
    aw, ab = lp['att']
    args = [GD, GS, h_Ee,
            w1[:H], w1[H:2 * H], w1[2 * H:], b1, w2, b2,
            ew1[:H], ew1[H:2 * H], ew1[2 * H:], eb1, ew2, eb2,
            aw[:H], aw[H:2 * H], aw[2 * H:], ab]
    specs = [_rows(_RE2, H), _rows(_RE2, H), _rows(_RE2, H)] + \
            [_full(a.shape) for a in args[3:]]
    return pl.pallas_call(
        _edgecomp_body,
        grid=(E // _RE2,),
        in_specs=specs,
        out_specs=[_rows(_RE2, H), _rows(_RE2, H), _rows(_RE2, 1),
                   pl.BlockSpec((1, 1), lambda i: (0, 0))],
        out_shape=[jax.ShapeDtypeStruct((E, H), jnp.float32),
                   jax.ShapeDtypeStruct((E, H), jnp.float32),
                   jax.ShapeDtypeStruct((E, 1), jnp.float32),
                   jax.ShapeDtypeStruct((1, 1), jnp.float32)],
    )(*args)


_NW = 32
_C = 128
_NCH = E // _C
_CPW = -(-_NCH // _NW)
_RPT = T // 16


def _sc_gather(G, dst, src):
    mesh = plsc.VectorSubcoreMesh(core_axis_name="c", subcore_axis_name="s")

    @functools.partial(
        pl.kernel,
        out_type=[jax.ShapeDtypeStruct((E, H), jnp.float32),
                  jax.ShapeDtypeStruct((E, H), jnp.float32)],
        mesh=mesh,
        compiler_params=pltpu.CompilerParams(use_tc_tiling_on_sc=True),
        scratch_types=[pltpu.VMEM((_C,), jnp.int32),
                       pltpu.VMEM((_C,), jnp.int32),
                       pltpu.VMEM((_C, H), jnp.float32),
                       pltpu.VMEM((_C, H), jnp.float32),
                       pltpu.SemaphoreType.DMA,
                       pltpu.SemaphoreType.DMA],
    )
    def k(g_hbm, dst_hbm, src_hbm, gd_hbm, gs_hbm, idxd, idxs, rowd, rows_, semd, sems):
        w = lax.axis_index("s") * 2 + lax.axis_index("c")

        def body(j, _):
            cw = w + j * _NW

            @pl.when(cw < _NCH)
            def _():
                base = cw * _C
                pltpu.sync_copy(dst_hbm.at[pl.ds(base, _C)], idxd)
                pltpu.sync_copy(src_hbm.at[pl.ds(base, _C)], idxs)
                cpd = pltpu.async_copy(g_hbm.at[idxd], rowd, semd)
                cps = pltpu.async_copy(g_hbm.at[idxs], rows_, sems)
                cpd.wait()
                cps.wait()
                pltpu.sync_copy(rowd, gd_hbm.at[pl.ds(base, _C)])
                pltpu.sync_copy(rows_, gs_hbm.at[pl.ds(base, _C)])
            return _

        lax.fori_loop(0, _CPW, body, None)

    return k(G, dst, src)


def _sc_scatter(S, dst, zrows, width, tc_tiling):
    mesh = plsc.VectorSubcoreMesh(core_axis_name="c", subcore_axis_name="s")

    @functools.partial(
        pl.kernel,
        out_type=jax.ShapeDtypeStruct((2, T, width), jnp.float32),
        mesh=mesh,
        compiler_params=pltpu.CompilerParams(use_tc_tiling_on_sc=tc_tiling),
        scratch_types=[pltpu.VMEM((_C,), jnp.int32),
                       pltpu.VMEM((_C, width), jnp.float32),
                       pltpu.VMEM_SHARED((T, width), jnp.float32)],
    )
    def k(s_hbm, dst_hbm, z_hbm, out_hbm, idxv, rowv, acc):
        c = lax.axis_index("c")
        s = lax.axis_index("s")
        w = s * 2 + c
        if tc_tiling:
            pltpu.sync_copy(z_hbm.at[pl.ds(0, 624)], acc.at[pl.ds(s * 624, 624)])

            @pl.when(s == 0)
            def _():
                pltpu.sync_copy(z_hbm.at[pl.ds(0, 16)], acc.at[pl.ds(9984, 16)])
        else:
            pltpu.sync_copy(z_hbm, acc.at[pl.ds(s * _RPT, _RPT)])
        plsc.subcore_barrier()

        def body(j, _):
            cw = w + j * _NW

            @pl.when(cw < _NCH)
            def _():
                base = cw * _C
                pltpu.sync_copy(dst_hbm.at[pl.ds(base, _C)], idxv)
                pltpu.sync_copy(s_hbm.at[pl.ds(base, _C)], rowv)
                pltpu.sync_copy(rowv, acc.at[idxv], add=True)
            return _

        lax.fori_loop(0, _CPW, body, None)
        plsc.subcore_barrier()
        if tc_tiling:
            pltpu.sync_copy(acc.at[pl.ds(s * 624, 624)],
                            out_hbm.at[c, pl.ds(s * 624, 624)])

            @pl.when(s == 0)
            def _():
                pltpu.sync_copy(acc.at[pl.ds(9984, 16)],
                                out_hbm.at[c, pl.ds(9984, 16)])
        else:
            pltpu.sync_copy(acc.at[pl.ds(s * _RPT, _RPT)],
                            out_hbm.at[c, pl.ds(s * _RPT, _RPT)])

    return k(S, dst, zrows)


def _scale_body(msg, att, m, s_out, w16_out):
    w = jnp.exp(att[...] - m[0, 0])
    s_out[...] = msg[...] * w
    lane = jax.lax.broadcasted_iota(jnp.int32, (_RE2, 16), 1)
    w16_out[...] = w * (lane == 0).astype(jnp.float32)


def _scale(msg, att, M):
    return pl.pallas_call(
        _scale_body,
        grid=(E // _RE2,),
        in_specs=[_rows(_RE2, H), _rows(_RE2, 1), pl.BlockSpec((1, 1), lambda i: (0, 0))],
        out_specs=[_rows(_RE2, H), _rows(_RE2, 16)],
        out_shape=[jax.ShapeDtypeStruct((E, H), jnp.float32),
                   jax.ShapeDtypeStruct((E, 16), jnp.float32)],
    )(msg, att, M)


def kernel(design_embed, esm_embed, struct_embed, esmif_embed, design_confs, esm_confs, h_E, params, design_pred_ids, esm_pred_ids, E_idx, attention_mask, batch_id):
    de = design_embed.reshape(T, -1)
    ee = esm_embed.reshape(T, -1)
    se = struct_embed.reshape(T, -1)
    ie = esmif_embed.reshape(T, -1)
    dc = design_confs.reshape(T, 1)
    ec = esm_confs.reshape(T, 1)
    dids = design_pred_ids.reshape(T, 1).astype(jnp.int32)
    eids = esm_pred_ids.reshape(T, 1).astype(jnp.int32)

    inputs_embeds = _towers(de, ee, se, ie, dc, ec, dids, eids, params)
    h_V = inputs_embeds
    h_Ee = _edge_embed(h_E, params)

    src = E_idx[0].astype(jnp.int32)
    dst = E_idx[1].astype(jnp.int32)
    zerosP = jnp.zeros((T, H), jnp.float32)
    zerosW = jnp.zeros((T, 16), jnp.float32)
    z128 = jnp.zeros((_RPT, H), jnp.float32)
    z16 = jnp.zeros((_RPT, 16), jnp.float32)
    p0, p1 = zerosP, zerosP
    pw0, pw1 = zerosW, zerosW
    layers = params['layers']
    for li, lp in enumerate(layers):
        h_V = _update_build(h_V, p0, p1, pw0, pw1)
        GD, GS = _sc_gather(h_V, dst, src)
        msg, h_Ee, att, M = _edgecomp(GD, GS, h_Ee, lp)
        S1, W16 = _scale(msg, att, M)
        P = _sc_scatter(S1, dst, z128, H, True)
        Pw = _sc_scatter(W16, dst, z16, 16, False)
        p0, p1 = P[0], P[1]
        pw0, pw1 = Pw[0], Pw[1]
    h_V = _update_build(h_V, p0, p1, pw0, pw1)

    logp = _readout(h_V, inputs_embeds, dc, params)
    return logp.reshape(B, N, 33)


# speedup vs baseline: 1.3715x; 1.0076x over previous
import functools

import jax
import jax.numpy as jnp
from jax import lax
from jax.experimental import pallas as pl
from jax.experimental.pallas import tpu as pltpu
from jax.experimental.pallas import tpu_sc as plsc

B, N = 8, 1250
T = B * N
E = 320000
H = 128

_RT = 400
_RE = 2560


def _mm(x, w):
    return jax.lax.dot_general(x, w, (((1,), (0,)), ((), ())),
                               preferred_element_type=jnp.float32)


def _mmb(x, w):
    return jax.lax.dot_general(x.astype(jnp.bfloat16), w.astype(jnp.bfloat16),
                               (((1,), (0,)), ((), ())),
                               preferred_element_type=jnp.float32)


def _full(shape):
    return pl.BlockSpec(shape, lambda i: tuple(0 for _ in shape))


def _rows(blk, width=None):
    if width is None:
        return pl.BlockSpec((blk,), lambda i: (i,))
    return pl.BlockSpec((blk, width), lambda i: (i, 0))


def _towers_body(de, ee, se, ie, dc, ec, dids, eids, *refs):
    (dp_w1, dp_b1, dp_w2, dp_b2, dp_w3, dp_b3,
     ep_w1, ep_b1, ep_w2, ep_b2, ep_w3, ep_b3,
     sp_w1, sp_b1, sp_w2, sp_b2, sp_w3, sp_b3,
     ip_w1, ip_b1, ip_w2, ip_b2, ip_w3, ip_b3,
     cf_w1, cf_b1, cf_w2, cf_b2, cf_w3, cf_b3,
     ef_w1, ef_b1, ef_w2, ef_b2, ef_w3, ef_b3,
     dtab, etab, out) = refs

    def mlp3(x, w1, b1, w2, b2, w3, b3):
        h = jax.nn.relu(_mmb(x, w1[...]) + b1[...])
        h = jax.nn.relu(_mmb(h, w2[...]) + b2[...])
        return _mmb(h, w3[...]) + b3[...]

    gnn = mlp3(de[...], dp_w1, dp_b1, dp_w2, dp_b2, dp_w3, dp_b3)
    esm = mlp3(ee[...], ep_w1, ep_b1, ep_w2, ep_b2, ep_w3, ep_b3)
    gearnet = mlp3(se[...], sp_w1, sp_b1, sp_w2, sp_b2, sp_w3, sp_b3)
    esmif = mlp3(ie[...], ip_w1, ip_b1, ip_w2, ip_b2, ip_w3, ip_b3)

    d1h = (dids[...] == jax.lax.broadcasted_iota(jnp.int32, (_RT, 33), 1)).astype(jnp.float32)
    e1h = (eids[...] == jax.lax.broadcasted_iota(jnp.int32, (_RT, 33), 1)).astype(jnp.float32)
    gnn = gnn + _mm(d1h, dtab[...])
    esm = esm + _mm(e1h, etab[...])

    def conf_mlp(x, w1, b1, w2, b2, w3, b3):
        h = jax.nn.relu(x * w1[...] + b1[...])
        h = jax.nn.relu(_mm(h, w2[...]) + b2[...])
        return _mm(h, w3[...]) + b3[...]

    conf = jax.nn.sigmoid(conf_mlp(dc[...], cf_w1, cf_b1, cf_w2, cf_b2, cf_w3, cf_b3))
    esm_conf = conf_mlp(ec[...], ef_w1, ef_b1, ef_w2, ef_b2, ef_w3, ef_b3)
    out[...] = gnn * conf + esm * esm_conf + gearnet + esmif


def _towers(de, ee, se, ie, dc, ec, dids, eids, params):
    wlist = []
    specs = []
    for name in ('DesignProj', 'ESMProj', 'StructProj', 'ESMIFProj',
                 'DesignConf', 'ESMConf'):
        for (w, b) in params[name]:
            wlist += [w, b]
            specs += [_full(w.shape), _full(b.shape)]
    wlist += [params['DesignEmbedTab'], params['ESMEmbedTab']]
    specs += [_full((33, H)), _full((33, H))]
    grid = T // _RT
    return pl.pallas_call(
        _towers_body,
        grid=(grid,),
        in_specs=[_rows(_RT, 1280), _rows(_RT, 1280), _rows(_RT, 3072),
                  _rows(_RT, 512), _rows(_RT, 1), _rows(_RT, 1),
                  _rows(_RT, 1), _rows(_RT, 1)] + specs,
        out_specs=_rows(_RT, H),
        out_shape=jax.ShapeDtypeStruct((T, H), jnp.float32),
    )(de, ee, se, ie, dc, ec, dids, eids, *wlist)


def _edge_body(he, w1, b1, w2, b2, w3, b3, out):
    h = jax.nn.relu(_mmb(he[...], w1[...]) + b1[...])
    h = jax.nn.relu(_mmb(h, w2[...]) + b2[...])
    out[...] = (_mmb(h, w3[...]) + b3[...]).astype(jnp.bfloat16)


def _edge_embed(h_E, params):
    (w1, b1), (w2, b2), (w3, b3) = params['EdgeEmbed']
    return pl.pallas_call(
        _edge_body,
        grid=(E // _RE,),
        in_specs=[_rows(_RE, 448), _full(w1.shape), _full(b1.shape),
                  _full(w2.shape), _full(b2.shape), _full(w3.shape), _full(b3.shape)],
        out_specs=_rows(_RE, H),
        out_shape=jax.ShapeDtypeStruct((E, H), jnp.bfloat16),
    )(h_E, w1, b1, w2, b2, w3, b3)


def _readout_body(hv, ie_, dc, ro_w, ro_b,
                  m1_w1, m1_b1, m1_w2, m1_b2, m1_w3, m1_b3,
                  m2_w1, m2_b1, m2_w2, m2_b2, m2_w3, m2_b3, out):
    logits = _mm(hv[...], ro_w[...]) + ro_b[...]
    m = jnp.max(logits, axis=-1, keepdims=True)
    s = jnp.sum(jnp.exp(logits - m), axis=-1, keepdims=True)
    confs = 1.0 / s

    def gate(x, w1, b1, w2, b2, w3, b3):
        h = jax.nn.relu(x * w1[...] + b1[...])
        h = jax.nn.relu(_mm(h, w2[...]) + b2[...])
        return jax.nn.sigmoid(_mm(h, w3[...]) + b3[...])

    dcv = dc[...]
    g1 = gate(confs - dcv, m1_w1, m1_b1, m1_w2, m1_b2, m1_w3, m1_b3)
    g2 = gate(dcv - confs, m2_w1, m2_b1, m2_w2, m2_b2, m2_w3, m2_b3)
    hv2 = hv[...] * g1 + ie_[...] * g2
    l2 = _mm(hv2, ro_w[...]) + ro_b[...]
    m2 = jnp.max(l2, axis=-1, keepdims=True)
    lse = m2 + jnp.log(jnp.sum(jnp.exp(l2 - m2), axis=-1, keepdims=True))
    out[...] = l2 - lse


def _readout(h_V, inputs_embeds, dc, params):
    ro_w, ro_b = params['ReadOut']
    wlist = [ro_w, ro_b]
    specs = [_full(ro_w.shape), _full(ro_b.shape)]
    for name in ('MLP1', 'MLP2'):
        for (w, b) in params[name]:
            wlist += [w, b]
            specs += [_full(w.shape), _full(b.shape)]
    return pl.pallas_call(
        _readout_body,
        grid=(T // _RT,),
        in_specs=[_rows(_RT, H), _rows(_RT, H), _rows(_RT, 1)] + specs,
        out_specs=_rows(_RT, 33),
        out_shape=jax.ShapeDtypeStruct((T, 33), jnp.float32),
    )(h_V, inputs_embeds, dc, *wlist)


_GW = 144


def _update_body(hv, p0, p1, pw0, pw1, hv_out):
    num = p0[...] + p1[...]
    den = pw0[:, 0:1] + pw1[:, 0:1]
    hv_out[...] = hv[...] + num / (den + 1e-9)


def _update_build(h_V, p0, p1, pw0, pw1):
    return pl.pallas_call(
        _update_body,
        grid=(T // _RT,),
        in_specs=[_rows(_RT, H), _rows(_RT, H), _rows(_RT, H),
                  _rows(_RT, 16), _rows(_RT, 16)],
        out_specs=_rows(_RT, H),
        out_shape=jax.ShapeDtypeStruct((T, H), jnp.float32),
    )(h_V, p0, p1, pw0, pw1)


_RE2 = 2000


def _edgecomp_body(gd, gs, he, w1d, w1e, w1s, b1, w2, b2,
                   ew1d, ew1e, ew1s, eb1, ew2, eb2, awd, awe, aws, ab,
                   msg_out, he_out, att_out, m_out):
    i = pl.program_id(0)
    gdh = gd[...]
    gsh = gs[...]
    hev = he[...]
    m1 = _mmb(gdh, w1d[...]) + _mmb(hev, w1e[...]) + _mmb(gsh, w1s[...]) + b1[...]
    msg_out[...] = (_mmb(jax.nn.relu(m1), w2[...]) + b2[...]).astype(jnp.bfloat16)
    e1 = _mmb(gdh, ew1d[...]) + _mmb(hev, ew1e[...]) + _mmb(gsh, ew1s[...]) + eb1[...]
    he_out[...] = (hev.astype(jnp.float32) + _mmb(jax.nn.relu(e1), ew2[...]) + eb2[...]).astype(jnp.bfloat16)
    att = (_mm(gdh.astype(jnp.float32), awd[...]) + _mm(hev.astype(jnp.float32), awe[...]) + _mm(gsh.astype(jnp.float32), aws[...]) + ab[0])
    att_out[...] = att
    bm = jnp.full((1, 1), jnp.max(att))

    @pl.when(i == 0)
    def _():
        m_out[...] = bm

    @pl.when(i > 0)
    def _():
        m_out[...] = jnp.maximum(m_out[...], bm)


def _edgecomp(GD, GS, h_Ee, lp):
    (w1, b1), (w2, b2) = lp['msg']
    (ew1, eb1), (ew2, eb2) = lp['edge']
    aw, ab = lp['att']
    args = [GD, GS, h_Ee,
            w1[:H], w1[H:2 * H], w1[2 * H:], b1, w2, b2,
            ew1[:H], ew1[H:2 * H], ew1[2 * H:], eb1, ew2, eb2,
            aw[:H], aw[H:2 * H], aw[2 * H:], ab]
    specs = [_rows(_RE2, H), _rows(_RE2, H), _rows(_RE2, H)] + \
            [_full(a.shape) for a in args[3:]]
    return pl.pallas_call(
        _edgecomp_body,
        grid=(E // _RE2,),
        in_specs=specs,
        out_specs=[_rows(_RE2, H), _rows(_RE2, H), _rows(_RE2, 1),
                   pl.BlockSpec((1, 1), lambda i: (0, 0))],
        out_shape=[jax.ShapeDtypeStruct((E, H), jnp.bfloat16),
                   jax.ShapeDtypeStruct((E, H), jnp.bfloat16),
                   jax.ShapeDtypeStruct((E, 1), jnp.float32),
                   jax.ShapeDtypeStruct((1, 1), jnp.float32)],
    )(*args)


_NW = 32
_C = 128
_NCH = E // _C
_CPW = -(-_NCH // _NW)
_RPT = T // 16


def _sc_gather(G, dst, src):
    mesh = plsc.VectorSubcoreMesh(core_axis_name="c", subcore_axis_name="s")

    @functools.partial(
        pl.kernel,
        out_type=[jax.ShapeDtypeStruct((E, H), jnp.float32),
                  jax.ShapeDtypeStruct((E, H), jnp.float32)],
        mesh=mesh,
        compiler_params=pltpu.CompilerParams(use_tc_tiling_on_sc=True),
        scratch_types=[pltpu.VMEM((_C,), jnp.int32),
                       pltpu.VMEM((_C,), jnp.int32),
                       pltpu.VMEM((_C, H), jnp.float32),
                       pltpu.VMEM((_C, H), jnp.float32),
                       pltpu.SemaphoreType.DMA,
                       pltpu.SemaphoreType.DMA],
    )
    def k(g_hbm, dst_hbm, src_hbm, gd_hbm, gs_hbm, idxd, idxs, rowd, rows_, semd, sems):
        w = lax.axis_index("s") * 2 + lax.axis_index("c")

        def body(j, _):
            cw = w + j * _NW

            @pl.when(cw < _NCH)
            def _():
                base = cw * _C
                pltpu.sync_copy(dst_hbm.at[pl.ds(base, _C)], idxd)
                pltpu.sync_copy(src_hbm.at[pl.ds(base, _C)], idxs)
                cpd = pltpu.async_copy(g_hbm.at[idxd], rowd, semd)
                cps = pltpu.async_copy(g_hbm.at[idxs], rows_, sems)
                cpd.wait()
                cps.wait()
                pltpu.sync_copy(rowd, gd_hbm.at[pl.ds(base, _C)])
                pltpu.sync_copy(rows_, gs_hbm.at[pl.ds(base, _C)])
            return _

        lax.fori_loop(0, _CPW, body, None)

    return k(G, dst, src)


def _sc_scatter(S, dst, zrows, width, tc_tiling):
    mesh = plsc.VectorSubcoreMesh(core_axis_name="c", subcore_axis_name="s")

    @functools.partial(
        pl.kernel,
        out_type=jax.ShapeDtypeStruct((2, T, width), jnp.float32),
        mesh=mesh,
        compiler_params=pltpu.CompilerParams(use_tc_tiling_on_sc=tc_tiling),
        scratch_types=[pltpu.VMEM((_C,), jnp.int32),
                       pltpu.VMEM((_C, width), jnp.float32),
                       pltpu.VMEM_SHARED((T, width), jnp.float32)],
    )
    def k(s_hbm, dst_hbm, z_hbm, out_hbm, idxv, rowv, acc):
        c = lax.axis_index("c")
        s = lax.axis_index("s")
        w = s * 2 + c
        if tc_tiling:
            pltpu.sync_copy(z_hbm.at[pl.ds(0, 624)], acc.at[pl.ds(s * 624, 624)])

            @pl.when(s == 0)
            def _():
                pltpu.sync_copy(z_hbm.at[pl.ds(0, 16)], acc.at[pl.ds(9984, 16)])
        else:
            pltpu.sync_copy(z_hbm, acc.at[pl.ds(s * _RPT, _RPT)])
        plsc.subcore_barrier()

        def body(j, _):
            cw = w + j * _NW

            @pl.when(cw < _NCH)
            def _():
                base = cw * _C
                pltpu.sync_copy(dst_hbm.at[pl.ds(base, _C)], idxv)
                pltpu.sync_copy(s_hbm.at[pl.ds(base, _C)], rowv)
                pltpu.sync_copy(rowv, acc.at[idxv], add=True)
            return _

        lax.fori_loop(0, _CPW, body, None)
        plsc.subcore_barrier()
        if tc_tiling:
            pltpu.sync_copy(acc.at[pl.ds(s * 624, 624)],
                            out_hbm.at[c, pl.ds(s * 624, 624)])

            @pl.when(s == 0)
            def _():
                pltpu.sync_copy(acc.at[pl.ds(9984, 16)],
                                out_hbm.at[c, pl.ds(9984, 16)])
        else:
            pltpu.sync_copy(acc.at[pl.ds(s * _RPT, _RPT)],
                            out_hbm.at[c, pl.ds(s * _RPT, _RPT)])

    return k(S, dst, zrows)


def _scale_body(msg, att, m, s_out, w16_out):
    w = jnp.exp(att[...] - m[0, 0])
    s_out[...] = msg[...].astype(jnp.float32) * w
    lane = jax.lax.broadcasted_iota(jnp.int32, (_RE2, 16), 1)
    w16_out[...] = w * (lane == 0).astype(jnp.float32)


def _scale(msg, att, M):
    return pl.pallas_call(
        _scale_body,
        grid=(E // _RE2,),
        in_specs=[_rows(_RE2, H), _rows(_RE2, 1), pl.BlockSpec((1, 1), lambda i: (0, 0))],
        out_specs=[_rows(_RE2, H), _rows(_RE2, 16)],
        out_shape=[jax.ShapeDtypeStruct((E, H), jnp.float32),
                   jax.ShapeDtypeStruct((E, 16), jnp.float32)],
    )(msg, att, M)


def kernel(design_embed, esm_embed, struct_embed, esmif_embed, design_confs, esm_confs, h_E, params, design_pred_ids, esm_pred_ids, E_idx, attention_mask, batch_id):
    de = design_embed.reshape(T, -1)
    ee = esm_embed.reshape(T, -1)
    se = struct_embed.reshape(T, -1)
    ie = esmif_embed.reshape(T, -1)
    dc = design_confs.reshape(T, 1)
    ec = esm_confs.reshape(T, 1)
    dids = design_pred_ids.reshape(T, 1).astype(jnp.int32)
    eids = esm_pred_ids.reshape(T, 1).astype(jnp.int32)

    inputs_embeds = _towers(de, ee, se, ie, dc, ec, dids, eids, params)
    h_V = inputs_embeds
    h_Ee = _edge_embed(h_E, params)

    src = E_idx[0].astype(jnp.int32)
    dst = E_idx[1].astype(jnp.int32)
    zerosP = jnp.zeros((T, H), jnp.float32)
    zerosW = jnp.zeros((T, 16), jnp.float32)
    z128 = jnp.zeros((_RPT, H), jnp.float32)
    z16 = jnp.zeros((_RPT, 16), jnp.float32)
    p0, p1 = zerosP, zerosP
    pw0, pw1 = zerosW, zerosW
    layers = params['layers']
    for li, lp in enumerate(layers):
        h_V = _update_build(h_V, p0, p1, pw0, pw1)
        GD, GS = _sc_gather(h_V, dst, src)
        msg, h_Ee, att, M = _edgecomp(GD, GS, h_Ee, lp)
        S1, W16 = _scale(msg, att, M)
        P = _sc_scatter(S1, dst, z128, H, True)
        Pw = _sc_scatter(W16, dst, z16, 16, False)
        p0, p1 = P[0], P[1]
        pw0, pw1 = Pw[0], Pw[1]
    h_V = _update_build(h_V, p0, p1, pw0, pw1)

    logp = _readout(h_V, inputs_embeds, dc, params)
    return logp.reshape(B, N, 33)


# speedup vs baseline: 1.3729x; 1.0010x over previous
import functools

import jax
import jax.numpy as jnp
from jax import lax
from jax.experimental import pallas as pl
from jax.experimental.pallas import tpu as pltpu
from jax.experimental.pallas import tpu_sc as plsc

B, N = 8, 1250
T = B * N
E = 320000
H = 128

_RT = 400
_RE = 2560


def _mm(x, w):
    return jax.lax.dot_general(x, w, (((1,), (0,)), ((), ())),
                               preferred_element_type=jnp.float32)


def _mmb(x, w):
    return jax.lax.dot_general(x.astype(jnp.bfloat16), w.astype(jnp.bfloat16),
                               (((1,), (0,)), ((), ())),
                               preferred_element_type=jnp.float32)


def _full(shape):
    return pl.BlockSpec(shape, lambda i: tuple(0 for _ in shape))


def _rows(blk, width=None):
    if width is None:
        return pl.BlockSpec((blk,), lambda i: (i,))
    return pl.BlockSpec((blk, width), lambda i: (i, 0))


def _towers_body(de, ee, se, ie, dc, ec, dids, eids, *refs):
    (dp_w1, dp_b1, dp_w2, dp_b2, dp_w3, dp_b3,
     ep_w1, ep_b1, ep_w2, ep_b2, ep_w3, ep_b3,
     sp_w1, sp_b1, sp_w2, sp_b2, sp_w3, sp_b3,
     ip_w1, ip_b1, ip_w2, ip_b2, ip_w3, ip_b3,
     cf_w1, cf_b1, cf_w2, cf_b2, cf_w3, cf_b3,
     ef_w1, ef_b1, ef_w2, ef_b2, ef_w3, ef_b3,
     dtab, etab, out) = refs

    def mlp3(x, w1, b1, w2, b2, w3, b3):
        h = jax.nn.relu(_mmb(x, w1[...]) + b1[...])
        h = jax.nn.relu(_mmb(h, w2[...]) + b2[...])
        return _mmb(h, w3[...]) + b3[...]

    gnn = mlp3(de[...], dp_w1, dp_b1, dp_w2, dp_b2, dp_w3, dp_b3)
    esm = mlp3(ee[...], ep_w1, ep_b1, ep_w2, ep_b2, ep_w3, ep_b3)
    gearnet = mlp3(se[...], sp_w1, sp_b1, sp_w2, sp_b2, sp_w3, sp_b3)
    esmif = mlp3(ie[...], ip_w1, ip_b1, ip_w2, ip_b2, ip_w3, ip_b3)

    d1h = (dids[...] == jax.lax.broadcasted_iota(jnp.int32, (_RT, 33), 1)).astype(jnp.float32)
    e1h = (eids[...] == jax.lax.broadcasted_iota(jnp.int32, (_RT, 33), 1)).astype(jnp.float32)
    gnn = gnn + _mm(d1h, dtab[...])
    esm = esm + _mm(e1h, etab[...])

    def conf_mlp(x, w1, b1, w2, b2, w3, b3):
        h = jax.nn.relu(x * w1[...] + b1[...])
        h = jax.nn.relu(_mm(h, w2[...]) + b2[...])
        return _mm(h, w3[...]) + b3[...]

    conf = jax.nn.sigmoid(conf_mlp(dc[...], cf_w1, cf_b1, cf_w2, cf_b2, cf_w3, cf_b3))
    esm_conf = conf_mlp(ec[...], ef_w1, ef_b1, ef_w2, ef_b2, ef_w3, ef_b3)
    out[...] = gnn * conf + esm * esm_conf + gearnet + esmif


def _towers(de, ee, se, ie, dc, ec, dids, eids, params):
    wlist = []
    specs = []
    for name in ('DesignProj', 'ESMProj', 'StructProj', 'ESMIFProj',
                 'DesignConf', 'ESMConf'):
        for (w, b) in params[name]:
            wlist += [w, b]
            specs += [_full(w.shape), _full(b.shape)]
    wlist += [params['DesignEmbedTab'], params['ESMEmbedTab']]
    specs += [_full((33, H)), _full((33, H))]
    grid = T // _RT
    return pl.pallas_call(
        _towers_body,
        grid=(grid,),
        in_specs=[_rows(_RT, 1280), _rows(_RT, 1280), _rows(_RT, 3072),
                  _rows(_RT, 512), _rows(_RT, 1), _rows(_RT, 1),
                  _rows(_RT, 1), _rows(_RT, 1)] + specs,
        out_specs=_rows(_RT, H),
        out_shape=jax.ShapeDtypeStruct((T, H), jnp.float32),
    )(de, ee, se, ie, dc, ec, dids, eids, *wlist)


def _edge_body(he, w1, b1, w2, b2, w3, b3, out):
    h = jax.nn.relu(_mmb(he[...], w1[...]) + b1[...])
    h = jax.nn.relu(_mmb(h, w2[...]) + b2[...])
    out[...] = (_mmb(h, w3[...]) + b3[...]).astype(jnp.bfloat16)


def _edge_embed(h_E, params):
    (w1, b1), (w2, b2), (w3, b3) = params['EdgeEmbed']
    return pl.pallas_call(
        _edge_body,
        grid=(E // _RE,),
        in_specs=[_rows(_RE, 448), _full(w1.shape), _full(b1.shape),
                  _full(w2.shape), _full(b2.shape), _full(w3.shape), _full(b3.shape)],
        out_specs=_rows(_RE, H),
        out_shape=jax.ShapeDtypeStruct((E, H), jnp.bfloat16),
    )(h_E, w1, b1, w2, b2, w3, b3)


def _readout_body(hv, ie_, dc, ro_w, ro_b,
                  m1_w1, m1_b1, m1_w2, m1_b2, m1_w3, m1_b3,
                  m2_w1, m2_b1, m2_w2, m2_b2, m2_w3, m2_b3, out):
    logits = _mm(hv[...], ro_w[...]) + ro_b[...]
    m = jnp.max(logits, axis=-1, keepdims=True)
    s = jnp.sum(jnp.exp(logits - m), axis=-1, keepdims=True)
    confs = 1.0 / s

    def gate(x, w1, b1, w2, b2, w3, b3):
        h = jax.nn.relu(x * w1[...] + b1[...])
        h = jax.nn.relu(_mm(h, w2[...]) + b2[...])
        return jax.nn.sigmoid(_mm(h, w3[...]) + b3[...])

    dcv = dc[...]
    g1 = gate(confs - dcv, m1_w1, m1_b1, m1_w2, m1_b2, m1_w3, m1_b3)
    g2 = gate(dcv - confs, m2_w1, m2_b1, m2_w2, m2_b2, m2_w3, m2_b3)
    hv2 = hv[...] * g1 + ie_[...] * g2
    l2 = _mm(hv2, ro_w[...]) + ro_b[...]
    m2 = jnp.max(l2, axis=-1, keepdims=True)
    lse = m2 + jnp.log(jnp.sum(jnp.exp(l2 - m2), axis=-1, keepdims=True))
    out[...] = l2 - lse


def _readout(h_V, inputs_embeds, dc, params):
    ro_w, ro_b = params['ReadOut']
    wlist = [ro_w, ro_b]
    specs = [_full(ro_w.shape), _full(ro_b.shape)]
    for name in ('MLP1', 'MLP2'):
        for (w, b) in params[name]:
            wlist += [w, b]
            specs += [_full(w.shape), _full(b.shape)]
    return pl.pallas_call(
        _readout_body,
        grid=(T // _RT,),
        in_specs=[_rows(_RT, H), _rows(_RT, H), _rows(_RT, 1)] + specs,
        out_specs=_rows(_RT, 33),
        out_shape=jax.ShapeDtypeStruct((T, 33), jnp.float32),
    )(h_V, inputs_embeds, dc, *wlist)


def _update_body(hv, p0, p1, pw0, pw1, hv_out):
    num = p0[...] + p1[...]
    den = pw0[:, 0:1] + pw1[:, 0:1]
    hv_out[...] = hv[...] + num / (den + 1e-9)


def _update_build(h_V, p0, p1, pw0, pw1):
    return pl.pallas_call(
        _update_body,
        grid=(T // _RT,),
        in_specs=[_rows(_RT, H), _rows(_RT, H), _rows(_RT, H),
                  _rows(_RT, 16), _rows(_RT, 16)],
        out_specs=_rows(_RT, H),
        out_shape=jax.ShapeDtypeStruct((T, H), jnp.float32),
    )(h_V, p0, p1, pw0, pw1)


_RE2 = 2000


def _edgecomp_body(gd, gs, he, w1d, w1e, w1s, b1, w2, b2,
                   ew1d, ew1e, ew1s, eb1, ew2, eb2, awd, awe, aws, ab,
                   msg_out, he_out, att_out, m_out):
    i = pl.program_id(0)
    gdh = gd[...]
    gsh = gs[...]
    hev = he[...]
    m1 = _mmb(gdh, w1d[...]) + _mmb(hev, w1e[...]) + _mmb(gsh, w1s[...]) + b1[...]
    msg_out[...] = (_mmb(jax.nn.relu(m1), w2[...]) + b2[...]).astype(jnp.bfloat16)
    e1 = _mmb(gdh, ew1d[...]) + _mmb(hev, ew1e[...]) + _mmb(gsh, ew1s[...]) + eb1[...]
    he_out[...] = (hev.astype(jnp.float32) + _mmb(jax.nn.relu(e1), ew2[...]) + eb2[...]).astype(jnp.bfloat16)
    att = (_mm(gdh.astype(jnp.float32), awd[...]) + _mm(hev.astype(jnp.float32), awe[...]) + _mm(gsh.astype(jnp.float32), aws[...]) + ab[0])
    att_out[...] = att
    bm = jnp.full((1, 1), jnp.max(att))

    @pl.when(i == 0)
    def _():
        m_out[...] = bm

    @pl.when(i > 0)
    def _():
        m_out[...] = jnp.maximum(m_out[...], bm)


def _edgecomp(GD, GS, h_Ee, lp):
    (w1, b1), (w2, b2) = lp['msg']
    (ew1, eb1), (ew2, eb2) = lp['edge']
    aw, ab = lp['att']
    args = [GD, GS, h_Ee,
            w1[:H], w1[H:2 * H], w1[2 * H:], b1, w2, b2,
            ew1[:H], ew1[H:2 * H], ew1[2 * H:], eb1, ew2, eb2,
            aw[:H], aw[H:2 * H], aw[2 * H:], ab]
    specs = [_rows(_RE2, H), _rows(_RE2, H), _rows(_RE2, H)] + \
            [_full(a.shape) for a in args[3:]]
    return pl.pallas_call(
        _edgecomp_body,
        grid=(E // _RE2,),
        in_specs=specs,
        out_specs=[_rows(_RE2, H), _rows(_RE2, H), _rows(_RE2, 1),
                   pl.BlockSpec((1, 1), lambda i: (0, 0))],
        out_shape=[jax.ShapeDtypeStruct((E, H), jnp.bfloat16),
                   jax.ShapeDtypeStruct((E, H), jnp.bfloat16),
                   jax.ShapeDtypeStruct((E, 1), jnp.float32),
                   jax.ShapeDtypeStruct((1, 1), jnp.float32)],
    )(*args)


_NW = 32
_C = 128
_NCH = E // _C
_CPW = -(-_NCH // _NW)
_RPT = T // 16


def _sc_gather(G, dst, src):
    mesh = plsc.VectorSubcoreMesh(core_axis_name="c", subcore_axis_name="s")

    @functools.partial(
        pl.kernel,
        out_type=[jax.ShapeDtypeStruct((E, H), jnp.float32),
                  jax.ShapeDtypeStruct((E, H), jnp.float32)],
        mesh=mesh,
        compiler_params=pltpu.CompilerParams(use_tc_tiling_on_sc=True),
        scratch_types=[pltpu.VMEM((_C,), jnp.int32),
                       pltpu.VMEM((_C,), jnp.int32),
                       pltpu.VMEM((_C, H), jnp.float32),
                       pltpu.VMEM((_C, H), jnp.float32),
                       pltpu.SemaphoreType.DMA,
                       pltpu.SemaphoreType.DMA],
    )
    def k(g_hbm, dst_hbm, src_hbm, gd_hbm, gs_hbm, idxd, idxs, rowd, rows_, semd, sems):
        w = lax.axis_index("s") * 2 + lax.axis_index("c")

        def body(j, _):
            cw = w + j * _NW

            @pl.when(cw < _NCH)
            def _():
                base = cw * _C
                pltpu.sync_copy(dst_hbm.at[pl.ds(base, _C)], idxd)
                pltpu.sync_copy(src_hbm.at[pl.ds(base, _C)], idxs)
                cpd = pltpu.async_copy(g_hbm.at[idxd], rowd, semd)
                cps = pltpu.async_copy(g_hbm.at[idxs], rows_, sems)
                cpd.wait()
                cps.wait()
                pltpu.sync_copy(rowd, gd_hbm.at[pl.ds(base, _C)])
                pltpu.sync_copy(rows_, gs_hbm.at[pl.ds(base, _C)])
            return _

        lax.fori_loop(0, _CPW, body, None)

    return k(G, dst, src)


def _sc_scatter(S, dst, zrows, width, tc_tiling):
    mesh = plsc.VectorSubcoreMesh(core_axis_name="c", subcore_axis_name="s")

    @functools.partial(
        pl.kernel,
        out_type=jax.ShapeDtypeStruct((2, T, width), jnp.float32),
        mesh=mesh,
        compiler_params=pltpu.CompilerParams(use_tc_tiling_on_sc=tc_tiling),
        scratch_types=[pltpu.VMEM((_C,), jnp.int32),
                       pltpu.VMEM((_C, width), jnp.float32),
                       pltpu.VMEM_SHARED((T, width), jnp.float32)],
    )
    def k(s_hbm, dst_hbm, z_hbm, out_hbm, idxv, rowv, acc):
        c = lax.axis_index("c")
        s = lax.axis_index("s")
        w = s * 2 + c
        if tc_tiling:
            pltpu.sync_copy(z_hbm.at[pl.ds(0, 624)], acc.at[pl.ds(s * 624, 624)])

            @pl.when(s == 0)
            def _():
                pltpu.sync_copy(z_hbm.at[pl.ds(0, 16)], acc.at[pl.ds(9984, 16)])
        else:
            pltpu.sync_copy(z_hbm, acc.at[pl.ds(s * _RPT, _RPT)])
        plsc.subcore_barrier()

        def body(j, _):
            cw = w + j * _NW

            @pl.when(cw < _NCH)
            def _():
                base = cw * _C
                pltpu.sync_copy(dst_hbm.at[pl.ds(base, _C)], idxv)
                pltpu.sync_copy(s_hbm.at[pl.ds(base, _C)], rowv)
                pltpu.sync_copy(rowv, acc.at[idxv], add=True)
            return _

        lax.fori_loop(0, _CPW, body, None)
        plsc.subcore_barrier()
        if tc_tiling:
            pltpu.sync_copy(acc.at[pl.ds(s * 624, 624)],
                            out_hbm.at[c, pl.ds(s * 624, 624)])

            @pl.when(s == 0)
            def _():
                pltpu.sync_copy(acc.at[pl.ds(9984, 16)],
                                out_hbm.at[c, pl.ds(9984, 16)])
        else:
            pltpu.sync_copy(acc.at[pl.ds(s * _RPT, _RPT)],
                            out_hbm.at[c, pl.ds(s * _RPT, _RPT)])

    return k(S, dst, zrows)


def _scale_body(msg, att, m, s_out, w16_out):
    w = jnp.exp(att[...] - m[0, 0])
    s_out[...] = msg[...].astype(jnp.float32) * w
    lane = jax.lax.broadcasted_iota(jnp.int32, (_RE2, 16), 1)
    w16_out[...] = w * (lane == 0).astype(jnp.float32)


def _scale(msg, att, M):
    return pl.pallas_call(
        _scale_body,
        grid=(E // _RE2,),
        in_specs=[_rows(_RE2, H), _rows(_RE2, 1), pl.BlockSpec((1, 1), lambda i: (0, 0))],
        out_specs=[_rows(_RE2, H), _rows(_RE2, 16)],
        out_shape=[jax.ShapeDtypeStruct((E, H), jnp.float32),
                   jax.ShapeDtypeStruct((E, 16), jnp.float32)],
    )(msg, att, M)


def kernel(design_embed, esm_embed, struct_embed, esmif_embed, design_confs, esm_confs, h_E, params, design_pred_ids, esm_pred_ids, E_idx, attention_mask, batch_id):
    de = design_embed.reshape(T, -1)
    ee = esm_embed.reshape(T, -1)
    se = struct_embed.reshape(T, -1)
    ie = esmif_embed.reshape(T, -1)
    dc = design_confs.reshape(T, 1)
    ec = esm_confs.reshape(T, 1)
    dids = design_pred_ids.reshape(T, 1).astype(jnp.int32)
    eids = esm_pred_ids.reshape(T, 1).astype(jnp.int32)

    inputs_embeds = _towers(de, ee, se, ie, dc, ec, dids, eids, params)
    h_V = inputs_embeds
    h_Ee = _edge_embed(h_E, params)

    src = E_idx[0].astype(jnp.int32)
    dst = E_idx[1].astype(jnp.int32)
    zerosP = jnp.zeros((T, H), jnp.float32)
    zerosW = jnp.zeros((T, 16), jnp.float32)
    z128 = jnp.zeros((_RPT, H), jnp.float32)
    z16 = jnp.zeros((_RPT, 16), jnp.float32)
    p0, p1 = zerosP, zerosP
    pw0, pw1 = zerosW, zerosW
    layers = params['layers']
    for li, lp in enumerate(layers):
        h_V = _update_build(h_V, p0, p1, pw0, pw1)
        GD, GS = _sc_gather(h_V, dst, src)
        msg, h_Ee, att, M = _edgecomp(GD, GS, h_Ee, lp)
        S1, W16 = _scale(msg, att, M)
        P = _sc_scatter(S1, dst, z128, H, True)
        Pw = _sc_scatter(W16, dst, z16, 16, False)
        p0, p1 = P[0], P[1]
        pw0, pw1 = Pw[0], Pw[1]
    h_V = _update_build(h_V, p0, p1, pw0, pw1)

    logp = _readout(h_V, inputs_embeds, dc, params)
    return logp.reshape(B, N, 33)
